# baseline (speedup 1.0000x reference)
import functools

import jax
import jax.numpy as jnp
from jax import lax
from jax.experimental import pallas as pl
from jax.experimental.pallas import tpu as pltpu
from jax.experimental.pallas import tpu_sc as plsc

NC = 2
NSUB = 16
L = 16
NW = NC * NSUB


def _proj_body(h_ref, w_ref, a1_ref, pch_ref, p_ref):
    ch = pl.program_id(1)
    P = jnp.dot(h_ref[:], w_ref[0], preferred_element_type=jnp.float32)
    pch_ref[0] = P
    part = jnp.dot(P, a1_ref[0], preferred_element_type=jnp.float32)

    @pl.when(ch == 0)
    def _():
        p_ref[:] = part

    @pl.when(ch != 0)
    def _():
        p_ref[:] += part


def _proj(h, w, a1, C, Fc, nt):
    N, d = h.shape
    wc = w.reshape(d, C, Fc).transpose(1, 0, 2)
    a1c = a1.reshape(C, Fc, 1)
    return pl.pallas_call(
        _proj_body,
        grid=(N // nt, C),
        in_specs=[
            pl.BlockSpec((nt, d), lambda i, c: (i, 0)),
            pl.BlockSpec((1, d, Fc), lambda i, c: (c, 0, 0)),
            pl.BlockSpec((1, Fc, 1), lambda i, c: (c, 0, 0)),
        ],
        out_specs=[
            pl.BlockSpec((1, nt, Fc), lambda i, c: (c, i, 0)),
            pl.BlockSpec((nt, 1), lambda i, c: (i, 0)),
        ],
        out_shape=[
            jax.ShapeDtypeStruct((C, N, Fc), jnp.float32),
            jax.ShapeDtypeStruct((N, 1), jnp.float32),
        ],
    )(h, wc, a1c)


def _mv_body(h_ref, w_ref, o_ref):
    o_ref[:] = jnp.dot(h_ref[:], w_ref[:], preferred_element_type=jnp.float32)


def _mv(h, w, nt):
    N, d = h.shape
    k = w.shape[1]
    return pl.pallas_call(
        _mv_body,
        grid=(N // nt,),
        in_specs=[
            pl.BlockSpec((nt, d), lambda i: (i, 0)),
            pl.BlockSpec((d, k), lambda i: (0, 0)),
        ],
        out_specs=pl.BlockSpec((nt, k), lambda i: (i, 0)),
        out_shape=jax.ShapeDtypeStruct((N, k), jnp.float32),
    )(h, w)


def _lin_body(h_ref, w_ref, b_ref, o_ref):
    o_ref[:] = (jnp.dot(h_ref[:], w_ref[:], preferred_element_type=jnp.float32)
                + b_ref[0, 0])


def _lin(h, w, b, nt):
    N, d = h.shape
    return pl.pallas_call(
        _lin_body,
        grid=(N // nt,),
        in_specs=[
            pl.BlockSpec((nt, d), lambda i: (i, 0)),
            pl.BlockSpec((d, 1), lambda i: (0, 0)),
            pl.BlockSpec((1, 1), lambda i: (0, 0)),
        ],
        out_specs=pl.BlockSpec((nt, 1), lambda i: (i, 0)),
        out_shape=jax.ShapeDtypeStruct((N, 1), jnp.float32),
    )(h, w, b)


def _elu(x):
    return jnp.where(x > 0, x, jnp.exp(x) - 1.0)


def _combs_body(n_gat, Fc, *refs):
    h_ref, w_ref = refs[0], refs[1]
    out_ref = refs[-1]
    acc = jnp.dot(h_ref[:], w_ref[:], preferred_element_type=jnp.float32)
    for g in range(n_gat):
        u = refs[2 + g][0]
        den = u[:, Fc] + 1e-9
        acc = acc + u[:, :Fc] / den[:, None]
    out_ref[:] = _elu(acc)


def _combine_s(h, wself, gat_tabs, C, Fc, nt):
    N, d = h.shape
    Wtab = Fc + 16
    in_specs = [
        pl.BlockSpec((nt, d), lambda i, c: (i, 0)),
        pl.BlockSpec((d, Fc), lambda i, c: (0, c)),
    ]
    for _ in gat_tabs:
        in_specs.append(pl.BlockSpec((1, nt, Wtab), lambda i, c: (c, i, 0)))
    return pl.pallas_call(
        functools.partial(_combs_body, len(gat_tabs), Fc),
        grid=(N // nt, C),
        in_specs=in_specs,
        out_specs=pl.BlockSpec((nt, Fc), lambda i, c: (i, c)),
        out_shape=jax.ShapeDtypeStruct((N, C * Fc), jnp.float32),
    )(h, wself, *gat_tabs)


def _combS_body(n_gat, C, Fc, *refs):
    h_ref, w_ref = refs[0], refs[1]
    out_ref = refs[-1]
    acc = jnp.dot(h_ref[:], w_ref[:], preferred_element_type=jnp.float32)
    parts = []
    for c in range(C):
        col = None
        for g in range(n_gat):
            u = refs[2 + g][c]
            den = u[:, Fc] + 1e-9
            part = u[:, :Fc] / den[:, None]
            col = part if col is None else col + part
        parts.append(col)
    out_ref[:] = _elu(acc + jnp.concatenate(parts, axis=1))


def _combine_S(h, wself, gat_tabs, C, Fc):
    N, d = h.shape
    Wtab = Fc + 16
    in_specs = [
        pl.BlockSpec((N, d), lambda: (0, 0)),
        pl.BlockSpec((d, d), lambda: (0, 0)),
    ]
    for _ in gat_tabs:
        in_specs.append(pl.BlockSpec((C, N, Wtab), lambda: (0, 0, 0)))
    return pl.pallas_call(
        functools.partial(_combS_body, len(gat_tabs), C, Fc),
        in_specs=in_specs,
        out_specs=pl.BlockSpec((N, C * Fc), lambda: (0, 0)),
        out_shape=jax.ShapeDtypeStruct((N, C * Fc), jnp.float32),
    )(h, wself, *gat_tabs)


def _combw_body(C, Fc, h_ref, w_ref, u_ref, out_ref):
    acc = jnp.dot(h_ref[:], w_ref[:], preferred_element_type=jnp.float32)
    parts = []
    for c in range(C):
        u = u_ref[c]
        den = u[:, Fc] + 1e-9
        parts.append(u[:, :Fc] / den[:, None])
    out_ref[:] = _elu(acc + jnp.concatenate(parts, axis=1))


def _combine_w(h, wself, U, C, Fc, nt):
    N, d = h.shape
    Wtab = Fc + 16
    return pl.pallas_call(
        functools.partial(_combw_body, C, Fc),
        grid=(N // nt,),
        in_specs=[
            pl.BlockSpec((nt, d), lambda i: (i, 0)),
            pl.BlockSpec((d, d), lambda i: (0, 0)),
            pl.BlockSpec((C, nt, Wtab), lambda i: (0, i, 0)),
        ],
        out_specs=pl.BlockSpec((nt, C * Fc), lambda i: (i, 0)),
        out_shape=jax.ShapeDtypeStruct((N, C * Fc), jnp.float32),
    )(h, wself, U)


def _gat_sc(Pch, pvec, qvec, src, dst, Rtab, G=128):
    C, Nsrc, Fc = Pch.shape
    E_pad = src.shape[0]
    Wtab = Fc + 16
    R32 = Rtab // NW
    B2 = 4096
    n_blocks = E_pad // B2
    assert n_blocks % 2 == 0
    cap = E_pad // NW + 4096

    mesh = plsc.VectorSubcoreMesh(core_axis_name="c", subcore_axis_name="s")

    @functools.partial(
        pl.kernel,
        mesh=mesh,
        compiler_params=pltpu.CompilerParams(needs_layout_passes=False,
                                             use_tc_tiling_on_sc=False),
        out_type=jax.ShapeDtypeStruct((C, Rtab, Wtab), jnp.float32),
        scratch_types=[
            pltpu.VMEM((2, B2), jnp.int32),
            pltpu.VMEM((2, B2), jnp.int32),
            pltpu.VMEM((cap,), jnp.int32),
            pltpu.VMEM((cap,), jnp.int32),
            pltpu.VMEM((cap,), jnp.float32),
            pltpu.VMEM((G, Fc), jnp.float32),
            pltpu.VMEM((G, Fc), jnp.float32),
            pltpu.VMEM((G,), jnp.float32),
            pltpu.VMEM((G,), jnp.float32),
            pltpu.VMEM((G,), jnp.float32),
            pltpu.VMEM((G,), jnp.float32),
            pltpu.VMEM((R32, Wtab), jnp.float32),
            pltpu.SemaphoreType.DMA,
            pltpu.SemaphoreType.DMA,
            pltpu.SemaphoreType.DMA,
            pltpu.SemaphoreType.DMA,
            pltpu.SemaphoreType.DMA,
            pltpu.SemaphoreType.DMA,
        ],
    )
    def k(p_hbm, q_hbm, src_hbm, dst_hbm, ptab_hbm, u_hbm,
          sblk, dblk, sidx_c, didx_c, exall, rows_g0, rows_g1,
          tmpp0, tmpp1, tmpq0, tmpq1, subtab,
          gsem0, gsem1, psem0, psem1, qsem0, qsem1):
        cid = lax.axis_index("c")
        sid = lax.axis_index("s")
        wid = cid * NSUB + sid
        lo = wid * R32
        hi = lo + R32
        rows_g = (rows_g0, rows_g1)
        tmpp = (tmpp0, tmpp1)
        tmpq = (tmpq0, tmpq1)
        gsem = (gsem0, gsem1)
        psem = (psem0, psem1)
        qsem = (qsem0, qsem1)

        iota = lax.iota(jnp.int32, L)
        zv = jnp.zeros((L,), jnp.float32)
        zi = jnp.zeros((L,), jnp.int32)
        lov = jnp.full((L,), 1, jnp.int32) * lo

        def prez(i, _):
            sidx_c[pl.ds(i * L, L)] = zi
            didx_c[pl.ds(i * L, L)] = lov
            exall[pl.ds(i * L, L)] = zv
            return 0
        lax.fori_loop(0, cap // L, prez, 0)

        pltpu.async_copy(src_hbm.at[pl.ds(0, B2)], sblk.at[0], gsem0)
        pltpu.async_copy(dst_hbm.at[pl.ds(0, B2)], dblk.at[0], psem0)

        def scan_block(blk, n, b):
            @pl.when(blk + 1 < n_blocks)
            def _():
                off = (blk + 1) * B2
                pltpu.async_copy(src_hbm.at[pl.ds(off, B2)],
                                 sblk.at[1 - b], gsem[1 - b])
                pltpu.async_copy(dst_hbm.at[pl.ds(off, B2)],
                                 dblk.at[1 - b], psem[1 - b])
            off = blk * B2
            pltpu.make_async_copy(src_hbm.at[pl.ds(off, B2)],
                                  sblk.at[b], gsem[b]).wait()
            pltpu.make_async_copy(dst_hbm.at[pl.ds(off, B2)],
                                  dblk.at[b], psem[b]).wait()

            def scan16(j, n):
                sv = sblk[b, pl.ds(j * L, L)]
                dv = dblk[b, pl.ds(j * L, L)]
                m = (dv >= lo) & (dv < hi)
                mi = jnp.where(m, 1, 0).astype(jnp.int32)
                cum = plsc.cumsum(mi)
                pos = n + cum - mi
                msafe = m & (pos < cap)
                plsc.store_scatter(sidx_c, [pos], sv, mask=msafe)
                plsc.store_scatter(didx_c, [pos], dv, mask=msafe)
                return jnp.minimum(n + cum[L - 1], cap)
            return lax.fori_loop(0, B2 // L, scan16, n)

        def blk_pair(step, n):
            n = scan_block(2 * step, n, 0)
            n = scan_block(2 * step + 1, n, 1)
            return n
        n_match = lax.fori_loop(0, n_blocks // 2, blk_pair, 0)
        n_match = jnp.minimum(n_match, cap - 2 * G)
        pairs = (n_match + 2 * G - 1) // (2 * G)
        ngroups2 = pairs * 2

        def chunk_body(ch, _):
            def zrow(r, _):
                for kk in range(Wtab // L):
                    subtab[r, pl.ds(kk * L, L)] = zv
                return 0
            lax.fori_loop(0, R32, zrow, 0)

            @pl.when(pairs > 0)
            def _():
                g0 = sidx_c.at[pl.ds(0, G)]
                pltpu.async_copy(ptab_hbm.at[ch].at[g0], rows_g0, gsem0)

                @pl.when(ch == 0)
                def _():
                    pltpu.async_copy(p_hbm.at[g0], tmpp0, psem0)
                    pltpu.async_copy(q_hbm.at[didx_c.at[pl.ds(0, G)]],
                                     tmpq0, qsem0)

            def do_group(gi, b):
                @pl.when(gi + 1 < ngroups2)
                def _():
                    nxt = sidx_c.at[pl.ds((gi + 1) * G, G)]
                    pltpu.async_copy(ptab_hbm.at[ch].at[nxt],
                                     rows_g[1 - b], gsem[1 - b])

                    @pl.when(ch == 0)
                    def _():
                        pltpu.async_copy(p_hbm.at[nxt], tmpp[1 - b],
                                         psem[1 - b])
                        pltpu.async_copy(
                            q_hbm.at[didx_c.at[pl.ds((gi + 1) * G, G)]],
                            tmpq[1 - b], qsem[1 - b])

                cur = sidx_c.at[pl.ds(gi * G, G)]
                pltpu.make_async_copy(ptab_hbm.at[ch].at[cur],
                                      rows_g[b], gsem[b]).wait()

                @pl.when(ch == 0)
                def _():
                    pltpu.make_async_copy(p_hbm.at[cur],
                                          tmpp[b], psem[b]).wait()
                    pltpu.make_async_copy(
                        q_hbm.at[didx_c.at[pl.ds(gi * G, G)]],
                        tmpq[b], qsem[b]).wait()
                    for j in range(G // L):
                        pv = tmpp[b][pl.ds(j * L, L)]
                        qv = tmpq[b][pl.ds(j * L, L)]
                        z = pv + qv
                        e = jnp.where(z >= 0, z, 0.2 * z)
                        eidx = gi * G + j * L + iota
                        exall[pl.ds(gi * G + j * L, L)] = jnp.where(
                            eidx < n_match, jnp.exp(e), 0.0)

                def acc16(j, _):
                    base = gi * G + j * L
                    dv16 = didx_c[pl.ds(base, L)]
                    exv = exall[pl.ds(base, L)]
                    dl16 = dv16 - lov
                    for g in range(L):
                        row = j * L + g
                        dloc = dl16[g]
                        wv = jnp.full((L,), exv[g], jnp.float32)
                        for kk in range(Fc // L):
                            subtab[dloc, pl.ds(kk * L, L)] = (
                                subtab[dloc, pl.ds(kk * L, L)]
                                + rows_g[b][row, pl.ds(kk * L, L)] * wv)
                        subtab[dloc, pl.ds(Fc, L)] = (
                            subtab[dloc, pl.ds(Fc, L)] + wv)
                    return 0
                lax.fori_loop(0, G // L, acc16, 0)

            def pair_body(pp, _):
                do_group(2 * pp, 0)
                do_group(2 * pp + 1, 1)
                return 0
            lax.fori_loop(0, pairs, pair_body, 0)

            pltpu.sync_copy(subtab, u_hbm.at[ch, pl.ds(lo, R32)])
            return 0
        lax.fori_loop(0, C, chunk_body, 0)

    return k(pvec, qvec, src, dst, Pch)


def _pad_edges(edges, ndst, G):
    src, dst = edges[0], edges[1]
    E = src.shape[0]
    Ep = -(-E // (2 * NW * G)) * (2 * NW * G)
    pad = Ep - E
    src = jnp.concatenate([src, jnp.zeros((pad,), jnp.int32)])
    dst = jnp.concatenate([dst, jnp.full((pad,), ndst, jnp.int32)])
    return src, dst


def kernel(Hw, Hs, HS, s2w, w2s, s2s, S2s, s2S, S2S,
           W_w2s, a1_w2s, a2_w2s, W_s2s, a1_s2s, a2_s2s,
           W_S2s, a1_S2s, a2_S2s, Wself_s,
           W_s2w, a1_s2w, a2_s2w, Wself_w,
           W_s2S, a1_s2S, a2_s2S, W_S2S, a1_S2S, a2_S2S, Wself_S,
           Wlin, blin):
    HSp = jnp.pad(HS, ((0, 12), (0, 0)))

    w2s_se = _pad_edges(w2s, 10000, 128)
    s2s_se = _pad_edges(s2s, 10000, 128)
    S2s_se = _pad_edges(S2s, 10000, 128)
    s2w_se = _pad_edges(s2w, 50000, 128)
    s2S_se = _pad_edges(s2S, 500, 128)
    S2S_se = _pad_edges(S2S, 500, 128)

    def gat_full(Hsrc, qv, W, a1, se, rtab, C, Fc, nt_src):
        Pch, p = _proj(Hsrc, W, a1, C, Fc, nt_src)
        return _gat_sc(Pch, p.reshape(-1), qv, se[0], se[1], rtab)

    def s_update(Hw_c, Hs_c, HS_c):
        qs3 = _mv(Hs_c, jnp.stack([a2_w2s, a2_s2s, a2_S2s], axis=1), 2000)
        U1 = gat_full(Hw_c, qs3[:, 0], W_w2s, a1_w2s, w2s_se, 10112, 5, 128, 2000)
        U2 = gat_full(Hs_c, qs3[:, 1], W_s2s, a1_s2s, s2s_se, 10112, 5, 128, 2000)
        U3 = gat_full(HS_c, qs3[:, 2], W_S2s, a1_S2s, S2s_se, 10112, 5, 128, 512)
        return _combine_s(Hs_c, Wself_s, [U1, U2, U3], 5, 128, 2000)

    Hs1 = s_update(Hw, Hs, HSp)

    qw = _mv(Hw, a2_s2w.reshape(-1, 1), 2000)
    Uw = gat_full(Hs, qw[:, 0], W_s2w, a1_s2w, s2w_se, 50176, 8, 16, 2000)
    Hw1 = _combine_w(Hw, Wself_w, Uw, 8, 16, 2000)

    qS2 = _mv(HSp, jnp.stack([a2_s2S, a2_S2S], axis=1), 512)
    U4 = gat_full(Hs, qS2[:, 0], W_s2S, a1_s2S, s2S_se, 512, 4, 128, 2000)
    U5 = gat_full(HSp, qS2[:, 1], W_S2S, a1_S2S, S2S_se, 512, 4, 128, 512)
    HS1 = _combine_S(HSp, Wself_S, [U4, U5], 4, 128)

    Hs2 = s_update(Hw1, Hs1, HS1)
    out = _lin(Hs2, Wlin, blin.reshape(1, 1), 2000)
    return out.reshape(-1)


# speedup vs baseline: 1.0986x; 1.0986x over previous
import functools

import jax
import jax.numpy as jnp
from jax import lax
from jax.experimental import pallas as pl
from jax.experimental.pallas import tpu as pltpu
from jax.experimental.pallas import tpu_sc as plsc

NC = 2
NSUB = 16
L = 16
NW = NC * NSUB


def _proj_body(h_ref, w_ref, a1_ref, pch_ref, p_ref):
    ch = pl.program_id(1)
    P = jnp.dot(h_ref[:], w_ref[0], preferred_element_type=jnp.float32)
    pch_ref[0] = P
    part = jnp.dot(P, a1_ref[0], preferred_element_type=jnp.float32)

    @pl.when(ch == 0)
    def _():
        p_ref[:] = part

    @pl.when(ch != 0)
    def _():
        p_ref[:] += part


def _proj(h, w, a1, C, Fc, nt):
    N, d = h.shape
    wc = w.reshape(d, C, Fc).transpose(1, 0, 2)
    a1c = a1.reshape(C, Fc, 1)
    return pl.pallas_call(
        _proj_body,
        grid=(N // nt, C),
        in_specs=[
            pl.BlockSpec((nt, d), lambda i, c: (i, 0)),
            pl.BlockSpec((1, d, Fc), lambda i, c: (c, 0, 0)),
            pl.BlockSpec((1, Fc, 1), lambda i, c: (c, 0, 0)),
        ],
        out_specs=[
            pl.BlockSpec((1, nt, Fc), lambda i, c: (c, i, 0)),
            pl.BlockSpec((nt, 1), lambda i, c: (i, 0)),
        ],
        out_shape=[
            jax.ShapeDtypeStruct((C, N, Fc), jnp.float32),
            jax.ShapeDtypeStruct((N, 1), jnp.float32),
        ],
    )(h, wc, a1c)


def _mv_body(h_ref, w_ref, o_ref):
    o_ref[:] = jnp.dot(h_ref[:], w_ref[:], preferred_element_type=jnp.float32)


def _mv(h, w, nt):
    N, d = h.shape
    k = w.shape[1]
    return pl.pallas_call(
        _mv_body,
        grid=(N // nt,),
        in_specs=[
            pl.BlockSpec((nt, d), lambda i: (i, 0)),
            pl.BlockSpec((d, k), lambda i: (0, 0)),
        ],
        out_specs=pl.BlockSpec((nt, k), lambda i: (i, 0)),
        out_shape=jax.ShapeDtypeStruct((N, k), jnp.float32),
    )(h, w)


def _lin_body(h_ref, w_ref, b_ref, o_ref):
    o_ref[:] = (jnp.dot(h_ref[:], w_ref[:], preferred_element_type=jnp.float32)
                + b_ref[0, 0])


def _lin(h, w, b, nt):
    N, d = h.shape
    return pl.pallas_call(
        _lin_body,
        grid=(N // nt,),
        in_specs=[
            pl.BlockSpec((nt, d), lambda i: (i, 0)),
            pl.BlockSpec((d, 1), lambda i: (0, 0)),
            pl.BlockSpec((1, 1), lambda i: (0, 0)),
        ],
        out_specs=pl.BlockSpec((nt, 1), lambda i: (i, 0)),
        out_shape=jax.ShapeDtypeStruct((N, 1), jnp.float32),
    )(h, w, b)


def _elu(x):
    return jnp.where(x > 0, x, jnp.exp(x) - 1.0)


def _combs_body(n_gat, Fc, *refs):
    h_ref, w_ref = refs[0], refs[1]
    out_ref = refs[-1]
    acc = jnp.dot(h_ref[:], w_ref[:], preferred_element_type=jnp.float32)
    for g in range(n_gat):
        u = refs[2 + g][0]
        den = u[:, Fc] + 1e-9
        acc = acc + u[:, :Fc] / den[:, None]
    out_ref[:] = _elu(acc)


def _combine_s(h, wself, gat_tabs, C, Fc, nt):
    N, d = h.shape
    Wtab = Fc + 16
    in_specs = [
        pl.BlockSpec((nt, d), lambda i, c: (i, 0)),
        pl.BlockSpec((d, Fc), lambda i, c: (0, c)),
    ]
    for _ in gat_tabs:
        in_specs.append(pl.BlockSpec((1, nt, Wtab), lambda i, c: (c, i, 0)))
    return pl.pallas_call(
        functools.partial(_combs_body, len(gat_tabs), Fc),
        grid=(N // nt, C),
        in_specs=in_specs,
        out_specs=pl.BlockSpec((nt, Fc), lambda i, c: (i, c)),
        out_shape=jax.ShapeDtypeStruct((N, C * Fc), jnp.float32),
    )(h, wself, *gat_tabs)


def _combS_body(n_gat, C, Fc, *refs):
    h_ref, w_ref = refs[0], refs[1]
    out_ref = refs[-1]
    acc = jnp.dot(h_ref[:], w_ref[:], preferred_element_type=jnp.float32)
    parts = []
    for c in range(C):
        col = None
        for g in range(n_gat):
            u = refs[2 + g][c]
            den = u[:, Fc] + 1e-9
            part = u[:, :Fc] / den[:, None]
            col = part if col is None else col + part
        parts.append(col)
    out_ref[:] = _elu(acc + jnp.concatenate(parts, axis=1))


def _combine_S(h, wself, gat_tabs, C, Fc):
    N, d = h.shape
    Wtab = Fc + 16
    in_specs = [
        pl.BlockSpec((N, d), lambda: (0, 0)),
        pl.BlockSpec((d, d), lambda: (0, 0)),
    ]
    for _ in gat_tabs:
        in_specs.append(pl.BlockSpec((C, N, Wtab), lambda: (0, 0, 0)))
    return pl.pallas_call(
        functools.partial(_combS_body, len(gat_tabs), C, Fc),
        in_specs=in_specs,
        out_specs=pl.BlockSpec((N, C * Fc), lambda: (0, 0)),
        out_shape=jax.ShapeDtypeStruct((N, C * Fc), jnp.float32),
    )(h, wself, *gat_tabs)


def _combw_body(C, Fc, h_ref, w_ref, u_ref, out_ref):
    acc = jnp.dot(h_ref[:], w_ref[:], preferred_element_type=jnp.float32)
    parts = []
    for c in range(C):
        u = u_ref[c]
        den = u[:, Fc] + 1e-9
        parts.append(u[:, :Fc] / den[:, None])
    out_ref[:] = _elu(acc + jnp.concatenate(parts, axis=1))


def _combine_w(h, wself, U, C, Fc, nt):
    N, d = h.shape
    Wtab = Fc + 16
    return pl.pallas_call(
        functools.partial(_combw_body, C, Fc),
        grid=(N // nt,),
        in_specs=[
            pl.BlockSpec((nt, d), lambda i: (i, 0)),
            pl.BlockSpec((d, d), lambda i: (0, 0)),
            pl.BlockSpec((C, nt, Wtab), lambda i: (0, i, 0)),
        ],
        out_specs=pl.BlockSpec((nt, C * Fc), lambda i: (i, 0)),
        out_shape=jax.ShapeDtypeStruct((N, C * Fc), jnp.float32),
    )(h, wself, U)


def _gat_sc(gats, Rtab, G=128):
    C, _, Fc = gats[0][0].shape
    Wtab = Fc + 16
    R32 = Rtab // NW
    B2 = 4096
    L2 = L
    nrel = len(gats)
    E_pads = [g[3].shape[0] for g in gats]
    caps = [ep // NW + 4096 for ep in E_pads]
    cap_max = max(caps)
    for g in gats:
        assert g[0].shape[2] == Fc and g[0].shape[0] == C

    mesh = plsc.VectorSubcoreMesh(core_axis_name="c", subcore_axis_name="s")

    @functools.partial(
        pl.kernel,
        mesh=mesh,
        compiler_params=pltpu.CompilerParams(needs_layout_passes=False,
                                             use_tc_tiling_on_sc=False),
        out_type=[jax.ShapeDtypeStruct((C, Rtab, Wtab), jnp.float32)
                  for _ in range(nrel)],
        scratch_types=[
            pltpu.VMEM((2, B2), jnp.int32),
            pltpu.VMEM((2, B2), jnp.int32),
            pltpu.VMEM((cap_max,), jnp.int32),
            pltpu.VMEM((cap_max,), jnp.int32),
            pltpu.VMEM((cap_max,), jnp.float32),
            pltpu.VMEM((G, Fc), jnp.float32),
            pltpu.VMEM((G, Fc), jnp.float32),
            pltpu.VMEM((G,), jnp.float32),
            pltpu.VMEM((G,), jnp.float32),
            pltpu.VMEM((G,), jnp.float32),
            pltpu.VMEM((G,), jnp.float32),
            pltpu.VMEM((R32, Wtab), jnp.float32),
            pltpu.SemaphoreType.DMA,
            pltpu.SemaphoreType.DMA,
            pltpu.SemaphoreType.DMA,
            pltpu.SemaphoreType.DMA,
            pltpu.SemaphoreType.DMA,
            pltpu.SemaphoreType.DMA,
        ],
    )
    def k(*refs):
        ins = refs[:5 * nrel]
        outs = refs[5 * nrel:5 * nrel + nrel]
        (sblk, dblk, sidx_c, didx_c, exall, rows_g0, rows_g1,
         tmpp0, tmpp1, tmpq0, tmpq1, subtab,
         gsem0, gsem1, psem0, psem1, qsem0, qsem1) = refs[5 * nrel + nrel:]
        cid = lax.axis_index("c")
        sid = lax.axis_index("s")
        wid = cid * NSUB + sid
        lo = wid * R32
        hi = lo + R32
        rows_g = (rows_g0, rows_g1)
        tmpp = (tmpp0, tmpp1)
        tmpq = (tmpq0, tmpq1)
        gsem = (gsem0, gsem1)
        psem = (psem0, psem1)
        qsem = (qsem0, qsem1)

        iota = lax.iota(jnp.int32, L2)
        zv = jnp.zeros((L2,), jnp.float32)
        zi = jnp.zeros((L2,), jnp.int32)
        lov = jnp.full((L2,), 1, jnp.int32) * lo

        def run_rel(p_hbm, q_hbm, src_hbm, dst_hbm, ptab_hbm, u_hbm,
                    E_pad, cap):
            n_blocks = E_pad // B2

            def prez(i, _):
                sidx_c[pl.ds(i * L2, L2)] = zi
                didx_c[pl.ds(i * L2, L2)] = lov
                exall[pl.ds(i * L2, L2)] = zv
                return 0
            lax.fori_loop(0, cap // L2, prez, 0)

            pltpu.async_copy(src_hbm.at[pl.ds(0, B2)], sblk.at[0], gsem0)
            pltpu.async_copy(dst_hbm.at[pl.ds(0, B2)], dblk.at[0], psem0)

            def scan_block(blk, n, b):
                @pl.when(blk + 1 < n_blocks)
                def _():
                    off = (blk + 1) * B2
                    pltpu.async_copy(src_hbm.at[pl.ds(off, B2)],
                                     sblk.at[1 - b], gsem[1 - b])
                    pltpu.async_copy(dst_hbm.at[pl.ds(off, B2)],
                                     dblk.at[1 - b], psem[1 - b])
                off = blk * B2
                pltpu.make_async_copy(src_hbm.at[pl.ds(off, B2)],
                                      sblk.at[b], gsem[b]).wait()
                pltpu.make_async_copy(dst_hbm.at[pl.ds(off, B2)],
                                      dblk.at[b], psem[b]).wait()

                def scan16(j, n):
                    sv = sblk[b, pl.ds(j * L2, L2)]
                    dv = dblk[b, pl.ds(j * L2, L2)]
                    m = (dv >= lo) & (dv < hi)
                    mi = jnp.where(m, 1, 0).astype(jnp.int32)
                    cum = plsc.cumsum(mi)
                    pos = n + cum - mi
                    msafe = m & (pos < cap)
                    plsc.store_scatter(sidx_c, [pos], sv, mask=msafe)
                    plsc.store_scatter(didx_c, [pos], dv, mask=msafe)
                    return jnp.minimum(n + cum[L2 - 1], cap)
                return lax.fori_loop(0, B2 // L2, scan16, n)

            def blk_pair(step, n):
                n = scan_block(2 * step, n, 0)
                n = scan_block(2 * step + 1, n, 1)
                return n
            n_match = lax.fori_loop(0, n_blocks // 2, blk_pair, 0)
            n_match = jnp.minimum(n_match, cap - 2 * G)
            pairs = (n_match + 2 * G - 1) // (2 * G)
            ngroups2 = pairs * 2

            def chunk_body(ch, _):
                def zrow(r, _):
                    for kk in range(Wtab // L2):
                        subtab[r, pl.ds(kk * L2, L2)] = zv
                    return 0
                lax.fori_loop(0, R32, zrow, 0)

                @pl.when(pairs > 0)
                def _():
                    g0 = sidx_c.at[pl.ds(0, G)]
                    pltpu.async_copy(ptab_hbm.at[ch].at[g0], rows_g0, gsem0)

                    @pl.when(ch == 0)
                    def _():
                        pltpu.async_copy(p_hbm.at[g0], tmpp0, psem0)
                        pltpu.async_copy(q_hbm.at[didx_c.at[pl.ds(0, G)]],
                                         tmpq0, qsem0)

                def do_group(gi, b):
                    @pl.when(gi + 1 < ngroups2)
                    def _():
                        nxt = sidx_c.at[pl.ds((gi + 1) * G, G)]
                        pltpu.async_copy(ptab_hbm.at[ch].at[nxt],
                                         rows_g[1 - b], gsem[1 - b])

                        @pl.when(ch == 0)
                        def _():
                            pltpu.async_copy(p_hbm.at[nxt], tmpp[1 - b],
                                             psem[1 - b])
                            pltpu.async_copy(
                                q_hbm.at[didx_c.at[pl.ds((gi + 1) * G, G)]],
                                tmpq[1 - b], qsem[1 - b])

                    cur = sidx_c.at[pl.ds(gi * G, G)]
                    pltpu.make_async_copy(ptab_hbm.at[ch].at[cur],
                                          rows_g[b], gsem[b]).wait()

                    @pl.when(ch == 0)
                    def _():
                        pltpu.make_async_copy(p_hbm.at[cur],
                                              tmpp[b], psem[b]).wait()
                        pltpu.make_async_copy(
                            q_hbm.at[didx_c.at[pl.ds(gi * G, G)]],
                            tmpq[b], qsem[b]).wait()
                        for j in range(G // L2):
                            pv = tmpp[b][pl.ds(j * L2, L2)]
                            qv = tmpq[b][pl.ds(j * L2, L2)]
                            z = pv + qv
                            e = jnp.where(z >= 0, z, 0.2 * z)
                            eidx = gi * G + j * L2 + iota
                            exall[pl.ds(gi * G + j * L2, L2)] = jnp.where(
                                eidx < n_match, jnp.exp(e), 0.0)

                    def acc16(j, _):
                        base = gi * G + j * L2
                        dv16 = didx_c[pl.ds(base, L2)]
                        exv = exall[pl.ds(base, L2)]
                        dl16 = dv16 - lov
                        for g in range(L2):
                            row = j * L2 + g
                            dloc = dl16[g]
                            wv = jnp.full((L2,), exv[g], jnp.float32)
                            for kk in range(Fc // L2):
                                subtab[dloc, pl.ds(kk * L2, L2)] = (
                                    subtab[dloc, pl.ds(kk * L2, L2)]
                                    + rows_g[b][row, pl.ds(kk * L2, L2)] * wv)
                            subtab[dloc, pl.ds(Fc, L2)] = (
                                subtab[dloc, pl.ds(Fc, L2)] + wv)
                        return 0
                    lax.fori_loop(0, G // L2, acc16, 0)

                def pair_body(pp, _):
                    do_group(2 * pp, 0)
                    do_group(2 * pp + 1, 1)
                    return 0
                lax.fori_loop(0, pairs, pair_body, 0)

                pltpu.sync_copy(subtab, u_hbm.at[ch, pl.ds(lo, R32)])
                return 0
            lax.fori_loop(0, C, chunk_body, 0)

        for r in range(nrel):
            run_rel(ins[5 * r], ins[5 * r + 1], ins[5 * r + 2],
                    ins[5 * r + 3], ins[5 * r + 4], outs[r],
                    E_pads[r], caps[r])

    flat = []
    for (Pch, pvec, qvec, src, dst) in gats:
        flat += [pvec, qvec, src, dst, Pch]
    res = k(*flat)
    if not isinstance(res, (list, tuple)):
        res = [res]
    return list(res)


def _pad_edges(edges, ndst, G):
    src, dst = edges[0], edges[1]
    E = src.shape[0]
    Ep = -(-E // (2 * NW * G)) * (2 * NW * G)
    pad = Ep - E
    src = jnp.concatenate([src, jnp.zeros((pad,), jnp.int32)])
    dst = jnp.concatenate([dst, jnp.full((pad,), ndst, jnp.int32)])
    return src, dst


def kernel(Hw, Hs, HS, s2w, w2s, s2s, S2s, s2S, S2S,
           W_w2s, a1_w2s, a2_w2s, W_s2s, a1_s2s, a2_s2s,
           W_S2s, a1_S2s, a2_S2s, Wself_s,
           W_s2w, a1_s2w, a2_s2w, Wself_w,
           W_s2S, a1_s2S, a2_s2S, W_S2S, a1_S2S, a2_S2S, Wself_S,
           Wlin, blin):
    HSp = jnp.pad(HS, ((0, 12), (0, 0)))

    w2s_se = _pad_edges(w2s, 10000, 128)
    s2s_se = _pad_edges(s2s, 10000, 128)
    S2s_se = _pad_edges(S2s, 10000, 128)
    s2w_se = _pad_edges(s2w, 50000, 128)
    s2S_se = _pad_edges(s2S, 500, 128)
    S2S_se = _pad_edges(S2S, 500, 128)

    def prep(Hsrc, qv, W, a1, se, C, Fc, nt_src):
        Pch, p = _proj(Hsrc, W, a1, C, Fc, nt_src)
        return (Pch, p.reshape(-1), qv, se[0], se[1])

    def s_update(Hw_c, Hs_c, HS_c):
        qs3 = _mv(Hs_c, jnp.stack([a2_w2s, a2_s2s, a2_S2s], axis=1), 2000)
        g1 = prep(Hw_c, qs3[:, 0], W_w2s, a1_w2s, w2s_se, 5, 128, 2000)
        g2 = prep(Hs_c, qs3[:, 1], W_s2s, a1_s2s, s2s_se, 5, 128, 2000)
        g3 = prep(HS_c, qs3[:, 2], W_S2s, a1_S2s, S2s_se, 5, 128, 512)
        Us = _gat_sc([g1, g2, g3], 10112)
        return _combine_s(Hs_c, Wself_s, Us, 5, 128, 2000)

    Hs1 = s_update(Hw, Hs, HSp)

    qw = _mv(Hw, a2_s2w.reshape(-1, 1), 2000)
    gw = prep(Hs, qw[:, 0], W_s2w, a1_s2w, s2w_se, 4, 32, 2000)
    Uw = _gat_sc([gw], 50176)[0]
    Hw1 = _combine_w(Hw, Wself_w, Uw, 4, 32, 2000)

    qS2 = _mv(HSp, jnp.stack([a2_s2S, a2_S2S], axis=1), 512)
    g4 = prep(Hs, qS2[:, 0], W_s2S, a1_s2S, s2S_se, 4, 128, 2000)
    g5 = prep(HSp, qS2[:, 1], W_S2S, a1_S2S, S2S_se, 4, 128, 512)
    U45 = _gat_sc([g4, g5], 512)
    HS1 = _combine_S(HSp, Wself_S, U45, 4, 128)

    Hs2 = s_update(Hw1, Hs1, HS1)
    out = _lin(Hs2, Wlin, blin.reshape(1, 1), 2000)
    return out.reshape(-1)


# speedup vs baseline: 1.1522x; 1.0488x over previous
import functools

import jax
import jax.numpy as jnp
from jax import lax
from jax.experimental import pallas as pl
from jax.experimental.pallas import tpu as pltpu
from jax.experimental.pallas import tpu_sc as plsc

NC = 2
NSUB = 16
L = 16
NW = NC * NSUB


def _proj_body(h_ref, w_ref, a1_ref, pch_ref, p_ref):
    ch = pl.program_id(1)
    P = jnp.dot(h_ref[:], w_ref[0], preferred_element_type=jnp.float32)
    pch_ref[0] = P
    part = jnp.dot(P, a1_ref[0], preferred_element_type=jnp.float32)

    @pl.when(ch == 0)
    def _():
        p_ref[:] = part

    @pl.when(ch != 0)
    def _():
        p_ref[:] += part


def _proj(h, w, a1, C, Fc, nt):
    N, d = h.shape
    wc = w.reshape(d, C, Fc).transpose(1, 0, 2)
    a1c = a1.reshape(C, Fc, 1)
    return pl.pallas_call(
        _proj_body,
        grid=(N // nt, C),
        in_specs=[
            pl.BlockSpec((nt, d), lambda i, c: (i, 0)),
            pl.BlockSpec((1, d, Fc), lambda i, c: (c, 0, 0)),
            pl.BlockSpec((1, Fc, 1), lambda i, c: (c, 0, 0)),
        ],
        out_specs=[
            pl.BlockSpec((1, nt, Fc), lambda i, c: (c, i, 0)),
            pl.BlockSpec((nt, 1), lambda i, c: (i, 0)),
        ],
        out_shape=[
            jax.ShapeDtypeStruct((C, N, Fc), jnp.float32),
            jax.ShapeDtypeStruct((N, 1), jnp.float32),
        ],
    )(h, wc, a1c)


def _mv_body(h_ref, w_ref, o_ref):
    o_ref[:] = jnp.dot(h_ref[:], w_ref[:], preferred_element_type=jnp.float32)


def _mv(h, w, nt):
    N, d = h.shape
    k = w.shape[1]
    return pl.pallas_call(
        _mv_body,
        grid=(N // nt,),
        in_specs=[
            pl.BlockSpec((nt, d), lambda i: (i, 0)),
            pl.BlockSpec((d, k), lambda i: (0, 0)),
        ],
        out_specs=pl.BlockSpec((nt, k), lambda i: (i, 0)),
        out_shape=jax.ShapeDtypeStruct((N, k), jnp.float32),
    )(h, w)


def _lin_body(h_ref, w_ref, b_ref, o_ref):
    o_ref[:] = (jnp.dot(h_ref[:], w_ref[:], preferred_element_type=jnp.float32)
                + b_ref[0, 0])


def _lin(h, w, b, nt):
    N, d = h.shape
    return pl.pallas_call(
        _lin_body,
        grid=(N // nt,),
        in_specs=[
            pl.BlockSpec((nt, d), lambda i: (i, 0)),
            pl.BlockSpec((d, 1), lambda i: (0, 0)),
            pl.BlockSpec((1, 1), lambda i: (0, 0)),
        ],
        out_specs=pl.BlockSpec((nt, 1), lambda i: (i, 0)),
        out_shape=jax.ShapeDtypeStruct((N, 1), jnp.float32),
    )(h, w, b)


def _elu(x):
    return jnp.where(x > 0, x, jnp.exp(x) - 1.0)


def _combs_body(n_gat, Fc, *refs):
    h_ref, w_ref = refs[0], refs[1]
    out_ref = refs[-1]
    acc = jnp.dot(h_ref[:], w_ref[:], preferred_element_type=jnp.float32)
    halves = []
    for half in range(2):
        col = None
        for g in range(n_gat):
            u_ref = refs[2 + 2 * g + half]
            usum = u_ref[0, 0] + u_ref[1, 0]
            den = usum[:, Fc] + 1e-9
            part = usum[:, :Fc] / den[:, None]
            col = part if col is None else col + part
        halves.append(col)
    out_ref[:] = _elu(acc + jnp.concatenate(halves, axis=1))


def _combine_s(h, wself, gat_tabs, C, Fc, nt):
    N, d = h.shape
    Wtab = Fc + 16
    in_specs = [
        pl.BlockSpec((nt, d), lambda i, c: (i, 0)),
        pl.BlockSpec((d, 2 * Fc), lambda i, c: (0, c)),
    ]
    args = [h, wself]
    for U in gat_tabs:
        for half in range(2):
            in_specs.append(pl.BlockSpec(
                (2, 1, nt, Wtab), lambda i, c, hh=half: (0, 2 * c + hh, i, 0)))
        args += [U, U]
    return pl.pallas_call(
        functools.partial(_combs_body, len(gat_tabs), Fc),
        grid=(N // nt, C // 2),
        in_specs=in_specs,
        out_specs=pl.BlockSpec((nt, 2 * Fc), lambda i, c: (i, c)),
        out_shape=jax.ShapeDtypeStruct((N, C * Fc), jnp.float32),
    )(*args)


def _combS_body(n_gat, C, Fc, *refs):
    h_ref, w_ref = refs[0], refs[1]
    out_ref = refs[-1]
    acc = jnp.dot(h_ref[:], w_ref[:], preferred_element_type=jnp.float32)
    parts = []
    for c in range(C):
        col = None
        for g in range(n_gat):
            u = refs[2 + g][:]
            usum = u[0, c] + u[1, c]
            den = usum[:, Fc] + 1e-9
            part = usum[:, :Fc] / den[:, None]
            col = part if col is None else col + part
        parts.append(col)
    out_ref[:] = _elu(acc + jnp.concatenate(parts, axis=1))


def _combine_S(h, wself, gat_tabs, C, Fc):
    N, d = h.shape
    Wtab = Fc + 16
    in_specs = [
        pl.BlockSpec((N, d), lambda: (0, 0)),
        pl.BlockSpec((d, d), lambda: (0, 0)),
    ]
    for _ in gat_tabs:
        in_specs.append(pl.BlockSpec((2, C, N, Wtab), lambda: (0, 0, 0, 0)))
    return pl.pallas_call(
        functools.partial(_combS_body, len(gat_tabs), C, Fc),
        in_specs=in_specs,
        out_specs=pl.BlockSpec((N, C * Fc), lambda: (0, 0)),
        out_shape=jax.ShapeDtypeStruct((N, C * Fc), jnp.float32),
    )(h, wself, *gat_tabs)


def _combw_body(C, Fc, nb_half, h_ref, w_ref, u_ref, out_ref):
    acc = jnp.dot(h_ref[:], w_ref[:], preferred_element_type=jnp.float32)
    parts = []
    for c in range(C):
        u = u_ref[0, c]
        den = u[:, Fc] + 1e-9
        parts.append(u[:, :Fc] / den[:, None])
    out_ref[:] = _elu(acc + jnp.concatenate(parts, axis=1))


def _combine_w(h, wself, U, C, Fc, nt, nb_half):
    N, d = h.shape
    Wtab = Fc + 16
    return pl.pallas_call(
        functools.partial(_combw_body, C, Fc, nb_half),
        grid=(N // nt,),
        in_specs=[
            pl.BlockSpec((nt, d), lambda i: (i, 0)),
            pl.BlockSpec((d, d), lambda i: (0, 0)),
            pl.BlockSpec((1, C, nt, Wtab),
                         lambda i: (i // nb_half, 0, i % nb_half, 0)),
        ],
        out_specs=pl.BlockSpec((nt, C * Fc), lambda i: (i, 0)),
        out_shape=jax.ShapeDtypeStruct((N, C * Fc), jnp.float32),
    )(h, wself, U)


def _gat_sc(Pch, pvec, qvec, src, dst, Rtab, G, row_half=0):
    C, Nsrc, Fc = Pch.shape
    E_pad = src.shape[0]
    Wtab = Fc + 16
    n_split = NSUB if row_half else NW
    Et = E_pad // n_split
    n_groups = Et // G
    assert n_groups % 2 == 0
    rows_pt = Rtab // NSUB
    nzcopies = rows_pt // 8
    Ndst_q = qvec.shape[0]

    mesh = plsc.VectorSubcoreMesh(core_axis_name="c", subcore_axis_name="s")

    @functools.partial(
        pl.kernel,
        mesh=mesh,
        compiler_params=pltpu.CompilerParams(needs_layout_passes=False,
                                             use_tc_tiling_on_sc=False),
        out_type=jax.ShapeDtypeStruct((NC, C, Rtab, Wtab), jnp.float32),
        scratch_types=[
            pltpu.VMEM((Et,), jnp.int32),
            pltpu.VMEM((n_groups, G), jnp.int32),
            pltpu.VMEM((n_groups, G), jnp.float32),
            pltpu.VMEM((G, Fc), jnp.float32),
            pltpu.VMEM((G, Fc), jnp.float32),
            pltpu.VMEM((G, Wtab), jnp.float32),
            pltpu.VMEM((8, Wtab), jnp.float32),
            pltpu.VMEM((G,), jnp.float32),
            pltpu.VMEM((G,), jnp.float32),
            pltpu.VMEM((G,), jnp.float32),
            pltpu.VMEM((G,), jnp.float32),
            pltpu.VMEM_SHARED((Rtab, Wtab), jnp.float32),
            pltpu.SemaphoreType.DMA,
            pltpu.SemaphoreType.DMA,
            pltpu.SemaphoreType.DMA,
            pltpu.SemaphoreType.DMA,
            pltpu.SemaphoreType.DMA,
            pltpu.SemaphoreType.DMA,
        ],
    )
    def k(p_hbm, q_hbm, src_hbm, dst_hbm, ptab_hbm, u_hbm,
          sall, dall, exall, rows_g0, rows_g1, rows_s, zbuf,
          tmpp0, tmpp1, tmpq0, tmpq1, u_sh,
          gsem0, gsem1, psem0, psem1, qsem0, qsem1):
        cid = lax.axis_index("c")
        sid = lax.axis_index("s")
        if row_half:
            ebase = sid * Et
            dlo = cid * row_half
        else:
            ebase = (cid * NSUB + sid) * Et
            dlo = 0
        row0 = sid * rows_pt
        rows_g = (rows_g0, rows_g1)
        tmpp = (tmpp0, tmpp1)
        tmpq = (tmpq0, tmpq1)
        gsem = (gsem0, gsem1)
        psem = (psem0, psem1)
        qsem = (qsem0, qsem1)

        pltpu.sync_copy(src_hbm.at[pl.ds(ebase, Et)], sall)

        def dfire(gi, _):
            pltpu.async_copy(dst_hbm.at[pl.ds(ebase + gi * G, G)],
                             dall.at[gi], gsem0)
            return 0
        lax.fori_loop(0, n_groups, dfire, 0)

        def ddrain(gi, _):
            pltpu.make_async_copy(dst_hbm.at[pl.ds(ebase + gi * G, G)],
                                  dall.at[gi], gsem0).wait()
            return 0
        lax.fori_loop(0, n_groups, ddrain, 0)

        zer = jnp.zeros((L,), jnp.float32)

        def zrow(r, _):
            for kk in range(Wtab // L):
                zbuf[r, pl.ds(kk * L, L)] = zer
            return 0
        lax.fori_loop(0, 8, zrow, 0)

        def ztail(g, _):
            rows_s[g, pl.ds(Fc, L)] = zer
            return 0
        lax.fori_loop(0, G, ztail, 0)

        iota = lax.iota(jnp.int32, L)
        colFc = jnp.full((L,), Fc, jnp.int32)

        def sidx(gi):
            return sall.at[pl.ds(gi * G, G)]

        def chunk_body(ch, _):
            def zcopy(j, _):
                pltpu.sync_copy(zbuf, u_sh.at[pl.ds(row0 + j * 8, 8)])
                return 0
            lax.fori_loop(0, nzcopies, zcopy, 0)
            plsc.subcore_barrier()

            pltpu.async_copy(ptab_hbm.at[ch].at[sidx(0)], rows_g0, gsem0)

            @pl.when(ch == 0)
            def _():
                pltpu.async_copy(p_hbm.at[sidx(0)], tmpp0, psem0)
                pltpu.async_copy(q_hbm.at[dall.at[0]], tmpq0, qsem0)

            def do_group(gi, b):
                @pl.when(gi + 1 < n_groups)
                def _():
                    pltpu.async_copy(ptab_hbm.at[ch].at[sidx(gi + 1)],
                                     rows_g[1 - b], gsem[1 - b])

                    @pl.when(ch == 0)
                    def _():
                        pltpu.async_copy(p_hbm.at[sidx(gi + 1)],
                                         tmpp[1 - b], psem[1 - b])
                        pltpu.async_copy(q_hbm.at[dall.at[gi + 1]],
                                         tmpq[1 - b], qsem[1 - b])

                pltpu.make_async_copy(ptab_hbm.at[ch].at[sidx(gi)],
                                      rows_g[b], gsem[b]).wait()

                @pl.when(ch == 0)
                def _():
                    pltpu.make_async_copy(p_hbm.at[sidx(gi)],
                                          tmpp[b], psem[b]).wait()
                    pltpu.make_async_copy(q_hbm.at[dall.at[gi]],
                                          tmpq[b], qsem[b]).wait()
                    for j in range(G // L):
                        pv = tmpp[b][pl.ds(j * L, L)]
                        qv = tmpq[b][pl.ds(j * L, L)]
                        z = pv + qv
                        e = jnp.where(z >= 0, z, 0.2 * z)
                        exall[gi, pl.ds(j * L, L)] = jnp.exp(e)
                        if row_half:
                            dv = dall[gi, pl.ds(j * L, L)]
                            dloc = dv - dlo
                            ok = (dloc >= 0) & (dloc < row_half)
                            dall[gi, pl.ds(j * L, L)] = jnp.where(
                                ok, dloc, row_half)

                def scale(j, _):
                    exv = exall[gi, pl.ds(j * L, L)]
                    plsc.store_scatter(rows_s, [j * L + iota, colFc], exv)
                    for g in range(L):
                        wv = jnp.full((L,), exv[g], jnp.float32)
                        row = j * L + g
                        for kk in range(Fc // L):
                            rows_s[row, pl.ds(kk * L, L)] = (
                                rows_g[b][row, pl.ds(kk * L, L)] * wv)
                    return 0
                lax.fori_loop(0, G // L, scale, 0)
                pltpu.sync_copy(rows_s, u_sh.at[dall.at[gi]], add=True)

            def pair_body(step, _):
                do_group(2 * step, 0)
                do_group(2 * step + 1, 1)
                return 0
            lax.fori_loop(0, n_groups // 2, pair_body, 0)
            plsc.subcore_barrier()
            pltpu.sync_copy(u_sh.at[pl.ds(row0, rows_pt)],
                            u_hbm.at[cid, ch, pl.ds(row0, rows_pt)])
            return 0
        lax.fori_loop(0, C, chunk_body, 0)

    return k(pvec, qvec, src, dst, Pch)


def _pad_edges(edges, ndst, G):
    src, dst = edges[0], edges[1]
    E = src.shape[0]
    Ep = -(-E // (2 * NW * G)) * (2 * NW * G)
    pad = Ep - E
    src = jnp.concatenate([src, jnp.zeros((pad,), jnp.int32)])
    dst = jnp.concatenate([dst, jnp.full((pad,), ndst, jnp.int32)])
    return src, dst


def kernel(Hw, Hs, HS, s2w, w2s, s2s, S2s, s2S, S2S,
           W_w2s, a1_w2s, a2_w2s, W_s2s, a1_s2s, a2_s2s,
           W_S2s, a1_S2s, a2_S2s, Wself_s,
           W_s2w, a1_s2w, a2_s2w, Wself_w,
           W_s2S, a1_s2S, a2_s2S, W_S2S, a1_S2S, a2_S2S, Wself_S,
           Wlin, blin):
    HSp = jnp.pad(HS, ((0, 12), (0, 0)))

    w2s_se = _pad_edges(w2s, 10000, 128)
    s2s_se = _pad_edges(s2s, 10000, 128)
    S2s_se = _pad_edges(S2s, 10000, 128)
    s2w_se = _pad_edges(s2w, 50000, 128)
    s2S_se = _pad_edges(s2S, 500, 128)
    S2S_se = _pad_edges(S2S, 500, 128)

    def gat_full(Hsrc, qv, W, a1, se, rtab, C, Fc, G, nt_src, row_half=0):
        Pch, p = _proj(Hsrc, W, a1, C, Fc, nt_src)
        return _gat_sc(Pch, p.reshape(-1), qv, se[0], se[1], rtab, G,
                       row_half=row_half)

    def s_update(Hw_c, Hs_c, HS_c):
        qs3 = _mv(Hs_c, jnp.stack([a2_w2s, a2_s2s, a2_S2s], axis=1), 2000)
        U1 = gat_full(Hw_c, qs3[:, 0], W_w2s, a1_w2s, w2s_se, 10112, 10, 64, 128, 2000)
        U2 = gat_full(Hs_c, qs3[:, 1], W_s2s, a1_s2s, s2s_se, 10112, 10, 64, 128, 2000)
        U3 = gat_full(HS_c, qs3[:, 2], W_S2s, a1_S2s, S2s_se, 10112, 10, 64, 128, 512)
        return _combine_s(Hs_c, Wself_s, [U1, U2, U3], 10, 64, 2000)

    Hs1 = s_update(Hw, Hs, HSp)

    qw = _mv(Hw, a2_s2w.reshape(-1, 1), 2000)
    Uw = gat_full(Hs, qw[:, 0], W_s2w, a1_s2w, s2w_se, 25088, 8, 16, 128, 2000,
                  row_half=25000)
    Hw1 = _combine_w(Hw, Wself_w, Uw, 8, 16, 1000, 25)

    qS2 = _mv(HSp, jnp.stack([a2_s2S, a2_S2S], axis=1), 512)
    U4 = gat_full(Hs, qS2[:, 0], W_s2S, a1_s2S, s2S_se, 512, 4, 128, 128, 2000)
    U5 = gat_full(HSp, qS2[:, 1], W_S2S, a1_S2S, S2S_se, 512, 4, 128, 128, 512)
    HS1 = _combine_S(HSp, Wself_S, [U4, U5], 4, 128)

    Hs2 = s_update(Hw1, Hs1, HS1)
    out = _lin(Hs2, Wlin, blin.reshape(1, 1), 2000)
    return out.reshape(-1)
